# baseline (speedup 1.0000x reference)
import functools

import jax
import jax.numpy as jnp
from jax import lax
from jax.experimental import pallas as pl
from jax.experimental.pallas import tpu as pltpu
from jax.experimental.pallas import tpu_sc as plsc

NC = 2
NS = 16
NW = NC * NS
_SP = 56
_ECH = 16


def _worker_id():
    return lax.axis_index("s") * NC + lax.axis_index("c")


def _sc_mesh():
    return plsc.VectorSubcoreMesh(core_axis_name="c", subcore_axis_name="s")


def _mm_body(nf_ref, w_ref, o_ref):
    o_ref[...] = jnp.tanh(
        jnp.dot(nf_ref[...], w_ref[...], preferred_element_type=jnp.float32))


def _matmul_tanh(nf, w):
    n, d = nf.shape
    grid = 16
    blk = n // grid
    return pl.pallas_call(
        _mm_body,
        grid=(grid,),
        in_specs=[
            pl.BlockSpec((blk, d), lambda i: (i, 0)),
            pl.BlockSpec((d, d), lambda i: (0, 0)),
        ],
        out_specs=pl.BlockSpec((blk, d), lambda i: (i, 0)),
        out_shape=jax.ShapeDtypeStruct((n, d), jnp.float32),
    )(nf, w)


def _rows_body(npw, table, ids, out, idx_v, buf, sem):
    base = _worker_id() * npw
    pltpu.sync_copy(ids.at[pl.ds(base, npw)], idx_v)
    pltpu.async_copy(table.at[idx_v], buf, sem).wait()
    pltpu.sync_copy(buf, out.at[pl.ds(base, npw)])


def _gather_rows(table, ids):
    n = ids.shape[0]
    d = table.shape[1]
    npw = n // NW
    return pl.kernel(
        functools.partial(_rows_body, npw),
        jax.ShapeDtypeStruct((n, d), jnp.float32),
        mesh=_sc_mesh(),
        scratch_types=[
            pltpu.VMEM((npw,), jnp.int32),
            pltpu.VMEM((npw, d), jnp.float32),
            pltpu.SemaphoreType.DMA,
        ],
    )(table, ids)


def _main_body(epw, s, table, ids2, gat, src2, dst2, out,
               idx_v, sidx_v, didx_v, buf0, buf1, selb0, selb1,
               gsem, ssem0, ssem1, bsem):
    wid = _worker_id()
    exbase = wid * epw
    bufs = (buf0, buf1)
    ssems = (ssem0, ssem1)
    pltpu.sync_copy(ids2.at[pl.ds(exbase, epw)], idx_v)
    stores = [None, None]
    for e in range(epw):
        k = e % 2
        if stores[k] is not None:
            stores[k].wait()
        pltpu.async_copy(table.at[idx_v.at[e]], bufs[k], gsem).wait()
        stores[k] = pltpu.async_copy(
            bufs[k], out.at[pl.ds((exbase + e) * _SP, _SP)], ssems[k])
    stores[0].wait()
    stores[1].wait()
    nech = (epw * 8) // _ECH
    pltpu.sync_copy(src2.at[pl.ds(wid * nech, nech)], sidx_v)
    pltpu.sync_copy(dst2.at[pl.ds(wid * nech, nech)], didx_v)
    selbs = (selb0, selb1)
    scats = [None, None]
    for j in range(nech):
        k = j % 2
        if scats[k] is not None:
            scats[k].wait()
        pltpu.async_copy(gat.at[sidx_v.at[j]], selbs[k], gsem).wait()
        scats[k] = pltpu.async_copy(selbs[k], out.at[didx_v.at[j]], bsem)
    scats[0].wait()
    scats[1].wait()


def _main_gather_scatter(table, ids2, gat, src2, dst2):
    b, s = ids2.shape
    d = table.shape[1]
    epw = b // NW
    return pl.kernel(
        functools.partial(_main_body, epw, s),
        jax.ShapeDtypeStruct((b * _SP, d), jnp.float32),
        mesh=_sc_mesh(),
        scratch_types=[
            pltpu.VMEM((epw, _SP), jnp.int32),
            pltpu.VMEM((b * 8 // NW // _ECH, _ECH), jnp.int32),
            pltpu.VMEM((b * 8 // NW // _ECH, _ECH), jnp.int32),
            pltpu.VMEM((_SP, d), jnp.float32),
            pltpu.VMEM((_SP, d), jnp.float32),
            pltpu.VMEM((_ECH, d), jnp.float32),
            pltpu.VMEM((_ECH, d), jnp.float32),
            pltpu.SemaphoreType.DMA,
            pltpu.SemaphoreType.DMA,
            pltpu.SemaphoreType.DMA,
            pltpu.SemaphoreType.DMA,
        ],
    )(table, ids2, gat, src2, dst2)


def kernel(input_ids, sem_node_ids, sememe_positions, sememe_node_idx, table, W):
    b, s = input_ids.shape
    p = sememe_positions.shape[1]
    d = table.shape[1]

    ids2 = input_ids.astype(jnp.int32)
    pos = sememe_positions.astype(jnp.int32)
    nid = sememe_node_idx.astype(jnp.int32)

    eq = pos[:, :, None] == pos[:, None, :]
    parr = jnp.arange(p, dtype=jnp.int32)
    winner = jnp.max(jnp.where(eq, parr[None, None, :], -1), axis=-1)
    nid_w = jnp.take_along_axis(nid, winner, axis=1)

    ids56 = jnp.concatenate(
        [ids2, jnp.tile(ids2[:, -1:], (1, _SP - s))], axis=1)

    nf = _gather_rows(table, sem_node_ids.astype(jnp.int32))
    gat = _matmul_tanh(nf, W)

    halves = []
    h = b // 2
    for lo in (0, h):
        posh = pos[lo:lo + h]
        src2 = jnp.reshape(nid_w[lo:lo + h], (h * p // _ECH, _ECH))
        dst2 = jnp.reshape(
            jnp.arange(h, dtype=jnp.int32)[:, None] * _SP + posh,
            (h * p // _ECH, _ECH))
        lin = _main_gather_scatter(table, ids56[lo:lo + h], gat, src2, dst2)
        halves.append(lin.reshape(h, _SP, d)[:, :s, :])
    return jnp.concatenate(halves, axis=0)


# speedup vs baseline: 1.3656x; 1.3656x over previous
import functools

import jax
import jax.numpy as jnp
from jax import lax
from jax.experimental import pallas as pl
from jax.experimental.pallas import tpu as pltpu
from jax.experimental.pallas import tpu_sc as plsc

NC = 2
NS = 16
NW = NC * NS
_SP = 56
_ECH = 16


def _worker_id():
    return lax.axis_index("s") * NC + lax.axis_index("c")


def _sc_mesh():
    return plsc.VectorSubcoreMesh(core_axis_name="c", subcore_axis_name="s")


def _mm_body(nf_ref, w_ref, o_ref):
    o_ref[...] = jnp.tanh(
        jnp.dot(nf_ref[...], w_ref[...], preferred_element_type=jnp.float32))


def _matmul_tanh(nf, w):
    n, d = nf.shape
    grid = 16
    blk = n // grid
    return pl.pallas_call(
        _mm_body,
        grid=(grid,),
        in_specs=[
            pl.BlockSpec((blk, d), lambda i: (i, 0)),
            pl.BlockSpec((d, d), lambda i: (0, 0)),
        ],
        out_specs=pl.BlockSpec((blk, d), lambda i: (i, 0)),
        out_shape=jax.ShapeDtypeStruct((n, d), jnp.float32),
    )(nf, w)


def _rows_body(npw, table, ids, out, idx_v, buf, sem):
    base = _worker_id() * npw
    pltpu.sync_copy(ids.at[pl.ds(base, npw)], idx_v)
    pltpu.async_copy(table.at[idx_v], buf, sem).wait()
    pltpu.sync_copy(buf, out.at[pl.ds(base, npw)])


def _gather_rows(table, ids):
    n = ids.shape[0]
    d = table.shape[1]
    npw = n // NW
    return pl.kernel(
        functools.partial(_rows_body, npw),
        jax.ShapeDtypeStruct((n, d), jnp.float32),
        mesh=_sc_mesh(),
        scratch_types=[
            pltpu.VMEM((npw,), jnp.int32),
            pltpu.VMEM((npw, d), jnp.float32),
            pltpu.SemaphoreType.DMA,
        ],
    )(table, ids)


def _main_body(epw, s, table, ids2, gat, src2, dst2, out,
               idx_v, sidx_v, didx_v, buf0, buf1, selb0, selb1,
               gsem, ssem0, ssem1, bsem):
    wid = _worker_id()
    exbase = wid * epw
    bufs = (buf0, buf1)
    ssems = (ssem0, ssem1)
    pltpu.sync_copy(ids2.at[pl.ds(exbase, epw)], idx_v)
    stores = [None, None]
    for e in range(epw):
        k = e % 2
        if stores[k] is not None:
            stores[k].wait()
        pltpu.async_copy(table.at[idx_v.at[e]], bufs[k], gsem).wait()
        stores[k] = pltpu.async_copy(
            bufs[k], out.at[pl.ds((exbase + e) * _SP, _SP)], ssems[k])
    stores[0].wait()
    stores[1].wait()
    nech = (epw * 8) // _ECH
    pltpu.sync_copy(src2.at[pl.ds(wid * nech, nech)], sidx_v)
    pltpu.sync_copy(dst2.at[pl.ds(wid * nech, nech)], didx_v)
    selbs = (selb0, selb1)
    scats = [None, None]
    for j in range(nech):
        k = j % 2
        if scats[k] is not None:
            scats[k].wait()
        pltpu.async_copy(gat.at[sidx_v.at[j]], selbs[k], gsem).wait()
        scats[k] = pltpu.async_copy(selbs[k], out.at[didx_v.at[j]], bsem)
    scats[0].wait()
    scats[1].wait()


def _main_gather_scatter(table, ids2, gat, src2, dst2):
    b, s = ids2.shape
    d = table.shape[1]
    epw = b // NW
    return pl.kernel(
        functools.partial(_main_body, epw, s),
        jax.ShapeDtypeStruct((b * _SP, d), jnp.float32),
        mesh=_sc_mesh(),
        scratch_types=[
            pltpu.VMEM((epw, _SP), jnp.int32),
            pltpu.VMEM((b * 8 // NW // _ECH, _ECH), jnp.int32),
            pltpu.VMEM((b * 8 // NW // _ECH, _ECH), jnp.int32),
            pltpu.VMEM((_SP, d), jnp.float32),
            pltpu.VMEM((_SP, d), jnp.float32),
            pltpu.VMEM((_ECH, d), jnp.float32),
            pltpu.VMEM((_ECH, d), jnp.float32),
            pltpu.SemaphoreType.DMA,
            pltpu.SemaphoreType.DMA,
            pltpu.SemaphoreType.DMA,
            pltpu.SemaphoreType.DMA,
        ],
    )(table, ids2, gat, src2, dst2)


def kernel(input_ids, sem_node_ids, sememe_positions, sememe_node_idx, table, W):
    b, s = input_ids.shape
    p = sememe_positions.shape[1]
    d = table.shape[1]

    ids2 = input_ids.astype(jnp.int32)
    pos = sememe_positions.astype(jnp.int32)
    nid = sememe_node_idx.astype(jnp.int32)

    eq = pos[:, :, None] == pos[:, None, :]
    parr = jnp.arange(p, dtype=jnp.int32)
    winner = jnp.max(jnp.where(eq, parr[None, None, :], -1), axis=-1)
    src2 = jnp.take_along_axis(nid, winner, axis=1).reshape(b * p // _ECH, _ECH)
    dst2 = (jnp.arange(b, dtype=jnp.int32)[:, None] * _SP
            + pos).reshape(b * p // _ECH, _ECH)

    ids56 = jnp.concatenate(
        [ids2, jnp.tile(ids2[:, -1:], (1, _SP - s))], axis=1)

    nf = _gather_rows(table, sem_node_ids.astype(jnp.int32))
    gat = _matmul_tanh(nf, W)
    lin = _main_gather_scatter(table, ids56, gat, src2, dst2)
    return lin.reshape(b, _SP, d)[:, :s, :]
